# Initial kernel scaffold; baseline (speedup 1.0000x reference)
import jax
import jax.numpy as jnp
from jax.experimental import pallas as pl


def kernel(input_ids, token_weight, cultural_weight):
    raise NotImplementedError("write your pallas kernel here")


# speedup vs baseline: 6.1958x; 6.1958x over previous
import functools

import jax
import jax.numpy as jnp
from jax import lax
from jax.experimental import pallas as pl
from jax.experimental.pallas import tpu as pltpu
from jax.experimental.pallas import tpu_sc as plsc

EMBED_DIM = 128
NC, NS = 2, 16
NW = NC * NS
CHUNK = 128


@functools.cache
def _make_kernel(n_tokens: int):
    per_w = n_tokens // NW
    n_chunk = per_w // CHUNK
    n_pair = n_chunk // 2
    mesh = plsc.VectorSubcoreMesh(
        core_axis_name="c", subcore_axis_name="s",
        num_cores=NC, num_subcores=NS)

    @functools.partial(
        pl.kernel,
        out_type=jax.ShapeDtypeStruct((n_tokens, EMBED_DIM), jnp.float32),
        mesh=mesh,
        scratch_types=[
            pltpu.VMEM((n_chunk, CHUNK), jnp.int32),
            pltpu.VMEM((CHUNK, EMBED_DIM), jnp.float32),
            pltpu.VMEM((CHUNK, EMBED_DIM), jnp.float32),
            pltpu.SemaphoreType.DMA,
            pltpu.SemaphoreType.DMA,
            pltpu.SemaphoreType.DMA,
            pltpu.SemaphoreType.DMA,
        ],
    )
    def k(ids_hbm, cat_hbm, out_hbm, idx_v, buf0, buf1, g0, g1, w0, w1):
        wid = lax.axis_index("s") * NC + lax.axis_index("c")
        pltpu.sync_copy(ids_hbm.at[wid], idx_v)
        base = wid * per_w

        def gather(j, buf, sem):
            return pltpu.async_copy(cat_hbm.at[idx_v.at[j]], buf, sem)

        def write(j, buf, sem):
            return pltpu.async_copy(
                buf, out_hbm.at[pl.ds(base + j * CHUNK, CHUNK)], sem)

        gather(0, buf0, g0)

        @pl.loop(0, n_pair)
        def _(i):
            j0 = 2 * i
            pltpu.make_async_copy(cat_hbm.at[idx_v.at[j0]], buf0, g0).wait()

            @pl.when(i > 0)
            def _():
                pltpu.make_async_copy(
                    buf1, out_hbm.at[pl.ds(base, CHUNK)], w1).wait()

            gather(j0 + 1, buf1, g1)
            write(j0, buf0, w0)
            pltpu.make_async_copy(cat_hbm.at[idx_v.at[j0]], buf1, g1).wait()
            pltpu.make_async_copy(
                buf0, out_hbm.at[pl.ds(base, CHUNK)], w0).wait()

            @pl.when(i < n_pair - 1)
            def _():
                gather(j0 + 2, buf0, g0)

            write(j0 + 1, buf1, w1)

        pltpu.make_async_copy(buf1, out_hbm.at[pl.ds(base, CHUNK)], w1).wait()

    return k


def kernel(input_ids, token_weight, cultural_weight):
    b, s = input_ids.shape
    n = b * s
    cat = jnp.concatenate([token_weight, cultural_weight], axis=1)
    ids = input_ids.astype(jnp.int32).reshape(NW, n // (NW * CHUNK), CHUNK)
    out = _make_kernel(n)(ids, cat)
    return out.reshape(b, s, EMBED_DIM)


# speedup vs baseline: 7.6251x; 1.2307x over previous
import functools

import jax
import jax.numpy as jnp
from jax import lax
from jax.experimental import pallas as pl
from jax.experimental.pallas import tpu as pltpu
from jax.experimental.pallas import tpu_sc as plsc

EMBED_DIM = 128
NC, NS = 2, 16
NW = NC * NS
GROUP = 1


@functools.cache
def _make_kernel(b: int, s: int):
    rows_w = b // NW
    n_grp = rows_w // GROUP
    mesh = plsc.VectorSubcoreMesh(
        core_axis_name="c", subcore_axis_name="s",
        num_cores=NC, num_subcores=NS)

    @functools.partial(
        pl.kernel,
        out_type=jax.ShapeDtypeStruct((b, s, EMBED_DIM), jnp.float32),
        mesh=mesh,
        scratch_types=[
            pltpu.VMEM((rows_w, s), jnp.int32),
            pltpu.VMEM((s, EMBED_DIM), jnp.float32),
            pltpu.VMEM((s, EMBED_DIM), jnp.float32),
            pltpu.SemaphoreType.DMA,
            pltpu.SemaphoreType.DMA,
            pltpu.SemaphoreType.DMA,
            pltpu.SemaphoreType.DMA,
        ],
    )
    def k(ids_hbm, cat_hbm, out_hbm, idx_v, buf0, buf1, g0, g1, w0, w1):
        wid = lax.axis_index("s") * NC + lax.axis_index("c")
        row0 = wid * rows_w
        pltpu.sync_copy(ids_hbm.at[pl.ds(row0, rows_w)], idx_v)

        def gather(g, buf, sem):
            pltpu.async_copy(cat_hbm.at[idx_v.at[g]], buf, sem)

        def write(g, buf, sem):
            pltpu.async_copy(buf, out_hbm.at[row0 + g], sem)

        def gwait(buf, sem):
            pltpu.make_async_copy(cat_hbm.at[idx_v.at[0]], buf, sem).wait()

        def wwait(buf, sem):
            pltpu.make_async_copy(buf, out_hbm.at[row0], sem).wait()

        gather(0, buf0, g0)

        @pl.loop(0, n_grp // 2)
        def _(i):
            j0 = 2 * i
            gwait(buf0, g0)

            @pl.when(i > 0)
            def _():
                wwait(buf1, w1)

            gather(j0 + 1, buf1, g1)
            write(j0, buf0, w0)
            gwait(buf1, g1)
            wwait(buf0, w0)

            @pl.when(i < n_grp // 2 - 1)
            def _():
                gather(j0 + 2, buf0, g0)

            write(j0 + 1, buf1, w1)

        wwait(buf1, w1)

    return k


def kernel(input_ids, token_weight, cultural_weight):
    b, s = input_ids.shape
    cat = jnp.concatenate([token_weight, cultural_weight], axis=1)
    return _make_kernel(b, s)(input_ids.astype(jnp.int32), cat)


# speedup vs baseline: 15.3206x; 2.0092x over previous
import functools

import jax
import jax.numpy as jnp
from jax import lax
from jax.experimental import pallas as pl
from jax.experimental.pallas import tpu as pltpu
from jax.experimental.pallas import tpu_sc as plsc

EMBED_DIM = 128
NC, NS = 2, 16
NW = NC * NS
CHUNK = 128
GROUP = 2
BANK = GROUP * CHUNK


def _concat_body(tok_ref, cul_ref, out_ref):
    out_ref[...] = jnp.concatenate(
        [tok_ref[...].T, cul_ref[...].T], axis=1)


@functools.cache
def _make_concat(v: int, d_tok: int, d_cul: int):
    cols = 2048
    grid = (v + cols - 1) // cols
    return pl.pallas_call(
        _concat_body,
        grid=(grid,),
        in_specs=[
            pl.BlockSpec((d_tok, cols), lambda i: (0, i)),
            pl.BlockSpec((d_cul, cols), lambda i: (0, i)),
        ],
        out_specs=pl.BlockSpec((cols, d_tok + d_cul), lambda i: (i, 0)),
        out_shape=jax.ShapeDtypeStruct((v, d_tok + d_cul), jnp.float32),
    )


@functools.cache
def _make_gather(n_tokens: int):
    per_w = n_tokens // NW
    n_bank = per_w // BANK
    n_pair = n_bank // 2
    mesh = plsc.VectorSubcoreMesh(
        core_axis_name="c", subcore_axis_name="s",
        num_cores=NC, num_subcores=NS)

    @functools.partial(
        pl.kernel,
        out_type=jax.ShapeDtypeStruct((n_tokens, EMBED_DIM), jnp.float32),
        mesh=mesh,
        scratch_types=[
            pltpu.VMEM((per_w,), jnp.int32),
            pltpu.VMEM((BANK, EMBED_DIM), jnp.float32),
            pltpu.VMEM((BANK, EMBED_DIM), jnp.float32),
            pltpu.SemaphoreType.DMA,
            pltpu.SemaphoreType.DMA,
            pltpu.SemaphoreType.DMA,
            pltpu.SemaphoreType.DMA,
        ],
    )
    def k(ids_hbm, cat_hbm, out_hbm, idx_v, buf0, buf1, g0, g1, w0, w1):
        wid = lax.axis_index("s") * NC + lax.axis_index("c")
        base = wid * per_w
        pltpu.sync_copy(ids_hbm.at[pl.ds(base, per_w)], idx_v)

        def gather(bank, buf, sem):
            for c in range(GROUP):
                pltpu.async_copy(
                    cat_hbm.at[idx_v.at[pl.ds(bank * BANK + c * CHUNK, CHUNK)]],
                    buf.at[pl.ds(c * CHUNK, CHUNK)], sem)

        def write(bank, buf, sem):
            pltpu.async_copy(
                buf, out_hbm.at[pl.ds(base + bank * BANK, BANK)], sem)

        def gwait(buf, sem):
            for c in range(GROUP):
                pltpu.make_async_copy(
                    cat_hbm.at[idx_v.at[pl.ds(0, CHUNK)]],
                    buf.at[pl.ds(c * CHUNK, CHUNK)], sem).wait()

        def wwait(buf, sem):
            pltpu.make_async_copy(
                buf, out_hbm.at[pl.ds(base, BANK)], sem).wait()

        gather(0, buf0, g0)

        @pl.loop(0, n_pair)
        def _(i):
            j0 = 2 * i
            gwait(buf0, g0)

            @pl.when(i > 0)
            def _():
                wwait(buf1, w1)

            gather(j0 + 1, buf1, g1)
            write(j0, buf0, w0)
            gwait(buf1, g1)
            wwait(buf0, w0)
            gather(j0 + 2, buf0, g0)
            write(j0 + 1, buf1, w1)

        gwait(buf0, g0)
        wwait(buf1, w1)
        write(n_bank - 1, buf0, w0)
        wwait(buf0, w0)

    return k


def kernel(input_ids, token_weight, cultural_weight):
    b, s = input_ids.shape
    n = b * s
    v, d_tok = token_weight.shape
    d_cul = cultural_weight.shape[1]
    ids_flat = input_ids.T.astype(jnp.int32).reshape(n)
    cat = _make_concat(v, d_tok, d_cul)(token_weight.T, cultural_weight.T)
    out = _make_gather(n)(ids_flat, cat)
    return out.reshape(s, b, EMBED_DIM).transpose(1, 0, 2)


# speedup vs baseline: 15.5348x; 1.0140x over previous
import functools

import jax
import jax.numpy as jnp
from jax import lax
from jax.experimental import pallas as pl
from jax.experimental.pallas import tpu as pltpu
from jax.experimental.pallas import tpu_sc as plsc

EMBED_DIM = 128
NC, NS = 2, 16
NW = NC * NS
CHUNK = 128
GROUP = 2
BANK = GROUP * CHUNK


def _concat_body(tok_ref, cul_ref, out_ref):
    d = tok_ref.shape[0]
    i = lax.broadcasted_iota(jnp.int32, (d, d), 0)
    j = lax.broadcasted_iota(jnp.int32, (d, d), 1)
    eye = (i == j).astype(jnp.float32)
    dims = (((0,), (0,)), ((), ()))
    tok_t = lax.dot_general(tok_ref[...], eye, dims,
                            preferred_element_type=jnp.float32)
    cul_t = lax.dot_general(cul_ref[...], eye, dims,
                            preferred_element_type=jnp.float32)
    out_ref[...] = jnp.concatenate([tok_t, cul_t], axis=1)


@functools.cache
def _make_concat(v: int, d_tok: int, d_cul: int):
    cols = 2048
    grid = (v + cols - 1) // cols
    return pl.pallas_call(
        _concat_body,
        grid=(grid,),
        in_specs=[
            pl.BlockSpec((d_tok, cols), lambda i: (0, i)),
            pl.BlockSpec((d_cul, cols), lambda i: (0, i)),
        ],
        out_specs=pl.BlockSpec((cols, d_tok + d_cul), lambda i: (i, 0)),
        out_shape=jax.ShapeDtypeStruct((v, d_tok + d_cul), jnp.float32),
    )


@functools.cache
def _make_gather(n_tokens: int):
    per_w = n_tokens // NW
    n_bank = per_w // BANK
    n_round = (n_bank - 1) // 3
    mesh = plsc.VectorSubcoreMesh(
        core_axis_name="c", subcore_axis_name="s",
        num_cores=NC, num_subcores=NS)

    @functools.partial(
        pl.kernel,
        out_type=jax.ShapeDtypeStruct((n_tokens, EMBED_DIM), jnp.float32),
        mesh=mesh,
        scratch_types=[
            pltpu.VMEM((per_w,), jnp.int32),
            pltpu.VMEM((BANK, EMBED_DIM), jnp.float32),
            pltpu.VMEM((BANK, EMBED_DIM), jnp.float32),
            pltpu.VMEM((BANK, EMBED_DIM), jnp.float32),
            pltpu.SemaphoreType.DMA,
            pltpu.SemaphoreType.DMA,
            pltpu.SemaphoreType.DMA,
            pltpu.SemaphoreType.DMA,
            pltpu.SemaphoreType.DMA,
            pltpu.SemaphoreType.DMA,
        ],
    )
    def k(ids_hbm, cat_hbm, out_hbm, idx_v,
          buf0, buf1, buf2, g0, g1, g2, w0, w1, w2):
        wid = lax.axis_index("s") * NC + lax.axis_index("c")
        base = wid * per_w
        pltpu.sync_copy(ids_hbm.at[pl.ds(base, per_w)], idx_v)

        def gather(bank, buf, sem):
            for c in range(GROUP):
                pltpu.async_copy(
                    cat_hbm.at[idx_v.at[pl.ds(bank * BANK + c * CHUNK, CHUNK)]],
                    buf.at[pl.ds(c * CHUNK, CHUNK)], sem)

        def write(bank, buf, sem):
            pltpu.async_copy(
                buf, out_hbm.at[pl.ds(base + bank * BANK, BANK)], sem)

        def gwait(buf, sem):
            for c in range(GROUP):
                pltpu.make_async_copy(
                    cat_hbm.at[idx_v.at[pl.ds(0, CHUNK)]],
                    buf.at[pl.ds(c * CHUNK, CHUNK)], sem).wait()

        def wwait(buf, sem):
            pltpu.make_async_copy(
                buf, out_hbm.at[pl.ds(base, BANK)], sem).wait()

        gather(0, buf0, g0)
        gather(1, buf1, g1)

        @pl.loop(0, n_round)
        def _(i):
            t0 = 3 * i
            gwait(buf0, g0)

            @pl.when(i > 0)
            def _():
                wwait(buf2, w2)
            gather(t0 + 2, buf2, g2)
            write(t0, buf0, w0)
            gwait(buf1, g1)
            write(t0 + 1, buf1, w1)
            wwait(buf0, w0)
            gather(t0 + 3, buf0, g0)
            gwait(buf2, g2)
            write(t0 + 2, buf2, w2)
            wwait(buf1, w1)

            @pl.when(t0 + 4 < n_bank)
            def _():
                gather(t0 + 4, buf1, g1)

        gwait(buf0, g0)
        wwait(buf2, w2)
        write(n_bank - 1, buf0, w0)
        wwait(buf0, w0)

    return k


def kernel(input_ids, token_weight, cultural_weight):
    b, s = input_ids.shape
    n = b * s
    v, d_tok = token_weight.shape
    d_cul = cultural_weight.shape[1]
    ids_flat = input_ids.T.astype(jnp.int32).reshape(n)
    cat = _make_concat(v, d_tok, d_cul)(token_weight.T, cultural_weight.T)
    out = _make_gather(n)(ids_flat, cat)
    return out.reshape(s, b, EMBED_DIM).transpose(1, 0, 2)
